# baseline (speedup 1.0000x reference)
import functools

import jax
import jax.numpy as jnp
from jax import lax
from jax.experimental import pallas as pl
from jax.experimental.pallas import tpu as pltpu
from jax.experimental.pallas import tpu_sc as plsc

N_REAL = 10000
D = 128
DH = D // 2
E_REAL = 320000

NC = 2
NS = 16
NW = NC * NS
CHUNK = 128
N_CHUNKS = E_REAL // CHUNK
DEG_BASE = N_CHUNKS // NW
DEG_EXTRA = N_CHUNKS - DEG_BASE * NW
SCAT_BASE = N_CHUNKS // NS
SCAT_EXTRA = N_CHUNKS - SCAT_BASE * NS
N_P = 10240
ROWS_PER_TILE = N_P // NS
BLK = 1024
NBUF = 12

_mesh = plsc.VectorSubcoreMesh(core_axis_name="c", subcore_axis_name="s")


def _mm_nt(a, b):
    return lax.dot_general(a, b, (((1,), (1,)), ((), ())),
                           preferred_element_type=jnp.float32)


@functools.partial(
    pl.kernel,
    out_type=jax.ShapeDtypeStruct((NC, N_P), jnp.float32),
    mesh=_mesh,
    scratch_types=[
        pltpu.VMEM((DEG_BASE + 1, CHUNK), jnp.int32),
        pltpu.VMEM((CHUNK,), jnp.float32),
        pltpu.VMEM((ROWS_PER_TILE,), jnp.float32),
        pltpu.VMEM_SHARED((N_P,), jnp.float32),
    ],
    compiler_params=pltpu.CompilerParams(use_tc_tiling_on_sc=False),
)
def _sc_degree(ei_hbm, out_hbm, idx_v, ones_v, buf_v, acc_sh):
    c = lax.axis_index("c")
    s = lax.axis_index("s")
    wid = s * NC + c
    start = wid * DEG_BASE + jnp.minimum(wid, DEG_EXTRA)

    def fill16(i, _):
        ones_v[pl.ds(i * 16, 16)] = jnp.ones((16,), jnp.float32)
        return 0
    lax.fori_loop(0, CHUNK // 16, fill16, 0)

    def zero16(i, _):
        buf_v[pl.ds(i * 16, 16)] = jnp.zeros((16,), jnp.float32)
        return 0
    lax.fori_loop(0, ROWS_PER_TILE // 16, zero16, 0)

    pltpu.sync_copy(buf_v, acc_sh.at[pl.ds(s * ROWS_PER_TILE, ROWS_PER_TILE)])
    plsc.subcore_barrier()

    pltpu.sync_copy(ei_hbm.at[1, pl.ds(start, DEG_BASE)],
                    idx_v.at[pl.ds(0, DEG_BASE)])

    @pl.when(wid < DEG_EXTRA)
    def _():
        pltpu.sync_copy(ei_hbm.at[1, start + DEG_BASE], idx_v.at[DEG_BASE])

    def body(j, _):
        pltpu.sync_copy(ones_v, acc_sh.at[idx_v.at[j]], add=True)
        return 0
    lax.fori_loop(0, DEG_BASE, body, 0)

    @pl.when(wid < DEG_EXTRA)
    def _():
        pltpu.sync_copy(ones_v, acc_sh.at[idx_v.at[DEG_BASE]], add=True)

    plsc.subcore_barrier()
    rows = pl.ds(s * ROWS_PER_TILE, ROWS_PER_TILE)
    pltpu.sync_copy(acc_sh.at[rows], buf_v)
    pltpu.sync_copy(buf_v, out_hbm.at[c, rows])


def _fused_body(x_ref, w1_ref, b1_ref, wg_ref, d0_ref, d1_ref,
                glo_ref, ghi_ref, dinv_ref):
    h = jnp.maximum(_mm_nt(x_ref[...], w1_ref[...]) + b1_ref[...], 0.0)
    hw = _mm_nt(h, wg_ref[...])
    deg = d0_ref[...] + d1_ref[...] + 1.0
    dinv = lax.rsqrt(deg)
    dinv_ref[...] = dinv
    g = hw * dinv
    gb = g.astype(jnp.bfloat16)
    glo_ref[...] = gb[:, :DH]
    ghi_ref[...] = gb[:, DH:]


def _tc_fused(xp, W1, b1_2d, Wg, deg0, deg1):
    return pl.pallas_call(
        _fused_body,
        grid=(N_P // BLK,),
        in_specs=[
            pl.BlockSpec((BLK, D), lambda i: (i, 0)),
            pl.BlockSpec((D, D), lambda i: (0, 0)),
            pl.BlockSpec((1, D), lambda i: (0, 0)),
            pl.BlockSpec((D, D), lambda i: (0, 0)),
            pl.BlockSpec((BLK, 1), lambda i: (i, 0)),
            pl.BlockSpec((BLK, 1), lambda i: (i, 0)),
        ],
        out_specs=[
            pl.BlockSpec((BLK, DH), lambda i: (i, 0)),
            pl.BlockSpec((BLK, DH), lambda i: (i, 0)),
            pl.BlockSpec((BLK, 1), lambda i: (i, 0)),
        ],
        out_shape=[
            jax.ShapeDtypeStruct((N_P, DH), jnp.bfloat16),
            jax.ShapeDtypeStruct((N_P, DH), jnp.bfloat16),
            jax.ShapeDtypeStruct((N_P, 1), jnp.float32),
        ],
    )(xp, W1, b1_2d, Wg, deg0, deg1)


@functools.partial(
    pl.kernel,
    out_type=[
        jax.ShapeDtypeStruct((N_P, DH), jnp.bfloat16),
        jax.ShapeDtypeStruct((N_P, DH), jnp.bfloat16),
    ],
    mesh=_mesh,
    scratch_types=[
        pltpu.VMEM((SCAT_BASE + 1, CHUNK), jnp.int32),
        pltpu.VMEM((SCAT_BASE + 1, CHUNK), jnp.int32),
        pltpu.VMEM((NBUF, CHUNK, DH), jnp.bfloat16),
        pltpu.VMEM_SHARED((N_P, DH), jnp.bfloat16),
        [pltpu.SemaphoreType.DMA] * NBUF,
        [pltpu.SemaphoreType.DMA] * NBUF,
    ],
    compiler_params=pltpu.CompilerParams(use_tc_tiling_on_sc=False),
)
def _sc_scatter(ei_hbm, glo_hbm, ghi_hbm, outlo_hbm, outhi_hbm,
                src_v, dst_v, ring, acc_sh, gsems, ssems):
    c = lax.axis_index("c")
    s = lax.axis_index("s")
    start = s * SCAT_BASE + jnp.minimum(s, SCAT_EXTRA)
    cnt = SCAT_BASE + jnp.where(s < SCAT_EXTRA, 1, 0)

    def zero_row(i, _):
        ring[0, i // (DH // 32), pl.ds((i % (DH // 32)) * 32, 32)] = (
            jnp.zeros((32,), jnp.bfloat16))
        return 0
    lax.fori_loop(0, CHUNK * (DH // 32), zero_row, 0)

    def zero_acc(k, _):
        pltpu.sync_copy(ring.at[0],
                        acc_sh.at[pl.ds(s * ROWS_PER_TILE + k * CHUNK, CHUNK)])
        return 0
    lax.fori_loop(0, ROWS_PER_TILE // CHUNK, zero_acc, 0)
    plsc.subcore_barrier()

    pltpu.sync_copy(ei_hbm.at[0, pl.ds(start, SCAT_BASE)],
                    src_v.at[pl.ds(0, SCAT_BASE)])
    pltpu.sync_copy(ei_hbm.at[1, pl.ds(start, SCAT_BASE)],
                    dst_v.at[pl.ds(0, SCAT_BASE)])

    @pl.when(s < SCAT_EXTRA)
    def _():
        pltpu.sync_copy(ei_hbm.at[0, start + SCAT_BASE], src_v.at[SCAT_BASE])
        pltpu.sync_copy(ei_hbm.at[1, start + SCAT_BASE], dst_v.at[SCAT_BASE])

    def gather(g_hbm):
        for k in range(NBUF - 1):
            pltpu.async_copy(g_hbm.at[src_v.at[k]], ring.at[k], gsems[k])

        def body(jj, _):
            for k in range(NBUF):
                j = jj * NBUF + k
                m = (k + NBUF - 1) % NBUF
                pltpu.make_async_copy(g_hbm.at[src_v.at[j]], ring.at[k],
                                      gsems[k]).wait()

                @pl.when(jnp.logical_and(j >= 1, j + NBUF - 1 < cnt))
                def _():
                    pltpu.make_async_copy(ring.at[m],
                                          acc_sh.at[dst_v.at[j]],
                                          ssems[m]).wait()

                @pl.when(j + NBUF - 1 < cnt)
                def _():
                    pltpu.async_copy(g_hbm.at[src_v.at[j + NBUF - 1]],
                                     ring.at[m], gsems[m])

                pltpu.async_copy(ring.at[k], acc_sh.at[dst_v.at[j]],
                                 ssems[k], add=True)
            return 0
        lax.fori_loop(0, SCAT_BASE // NBUF, body, 0)

        @pl.when(cnt > SCAT_BASE)
        def _():
            pltpu.make_async_copy(g_hbm.at[src_v.at[SCAT_BASE]], ring.at[0],
                                  gsems[0]).wait()
            pltpu.async_copy(ring.at[0], acc_sh.at[dst_v.at[SCAT_BASE]],
                             ssems[0], add=True)

        for k in range(NBUF):
            pltpu.make_async_copy(ring.at[k], acc_sh.at[dst_v.at[0]],
                                  ssems[k]).wait()

    @pl.when(c == 0)
    def _():
        gather(glo_hbm)

    @pl.when(c == 1)
    def _():
        gather(ghi_hbm)

    plsc.subcore_barrier()

    def writeout(out_hbm):
        def wo(k, _):
            off = s * ROWS_PER_TILE + k * CHUNK
            pltpu.sync_copy(acc_sh.at[pl.ds(off, CHUNK)], ring.at[0])
            pltpu.sync_copy(ring.at[0], out_hbm.at[pl.ds(off, CHUNK)])
            return 0
        lax.fori_loop(0, ROWS_PER_TILE // CHUNK, wo, 0)

    @pl.when(c == 0)
    def _():
        writeout(outlo_hbm)

    @pl.when(c == 1)
    def _():
        writeout(outhi_hbm)


def _out_body(alo_ref, ahi_ref, glo_ref, ghi_ref, dinv_ref, bg_ref,
              w2_ref, b2_ref, o_ref):
    dinv = dinv_ref[...]
    alo = alo_ref[...].astype(jnp.float32)
    ahi = ahi_ref[...].astype(jnp.float32)
    glo = glo_ref[...].astype(jnp.float32)
    ghi = ghi_ref[...].astype(jnp.float32)
    conv = jnp.concatenate(
        [(alo + glo) * dinv,
         (ahi + ghi) * dinv], axis=1) + bg_ref[...]
    h2 = jnp.maximum(conv, 0.0)
    o_ref[...] = _mm_nt(h2, w2_ref[...]) + b2_ref[...]


OBLK = 2000


def _tc_out(alo, ahi, glo, ghi, dinv, bg_2d, W2, b2_2d):
    return pl.pallas_call(
        _out_body,
        grid=(N_REAL // OBLK,),
        in_specs=[
            pl.BlockSpec((OBLK, DH), lambda i: (i, 0)),
            pl.BlockSpec((OBLK, DH), lambda i: (i, 0)),
            pl.BlockSpec((OBLK, DH), lambda i: (i, 0)),
            pl.BlockSpec((OBLK, DH), lambda i: (i, 0)),
            pl.BlockSpec((OBLK, 1), lambda i: (i, 0)),
            pl.BlockSpec((1, D), lambda i: (0, 0)),
            pl.BlockSpec((D, D), lambda i: (0, 0)),
            pl.BlockSpec((1, D), lambda i: (0, 0)),
        ],
        out_specs=pl.BlockSpec((OBLK, D), lambda i: (i, 0)),
        out_shape=jax.ShapeDtypeStruct((N_REAL, D), jnp.float32),
    )(alo, ahi, glo, ghi, dinv, bg_2d, W2, b2_2d)


def kernel(x, edge_index, W1, b1, Wg, bg, W2, b2):
    ei3 = edge_index.astype(jnp.int32).reshape(2, N_CHUNKS, CHUNK)

    b1_2d = b1.reshape(1, D)
    bg_2d = bg.reshape(1, D)
    b2_2d = b2.reshape(1, D)

    deg_partials = _sc_degree(ei3)
    deg0 = deg_partials[0].reshape(N_P, 1)
    deg1 = deg_partials[1].reshape(N_P, 1)

    glo, ghi, dinv = _tc_fused(x, W1, b1_2d, Wg, deg0, deg1)

    alo, ahi = _sc_scatter(ei3, glo, ghi)

    return _tc_out(alo, ahi, glo, ghi, dinv, bg_2d, W2, b2_2d)


# speedup vs baseline: 1.0255x; 1.0255x over previous
import functools

import jax
import jax.numpy as jnp
from jax import lax
from jax.experimental import pallas as pl
from jax.experimental.pallas import tpu as pltpu
from jax.experimental.pallas import tpu_sc as plsc

N_REAL = 10000
D = 128
DH = D // 2
E_REAL = 320000

NC = 2
NS = 16
NW = NC * NS
CHUNK = 128
N_CHUNKS = E_REAL // CHUNK
DEG_BASE = N_CHUNKS // NW
DEG_EXTRA = N_CHUNKS - DEG_BASE * NW
SCAT_BASE = N_CHUNKS // NS
SCAT_EXTRA = N_CHUNKS - SCAT_BASE * NS
N_P = 10240
ROWS_PER_TILE = N_P // NS
BLK = 1024
NBUF = 6

_mesh = plsc.VectorSubcoreMesh(core_axis_name="c", subcore_axis_name="s")


def _mm_nt(a, b):
    return lax.dot_general(a, b, (((1,), (1,)), ((), ())),
                           preferred_element_type=jnp.float32)


@functools.partial(
    pl.kernel,
    out_type=jax.ShapeDtypeStruct((NC, N_P), jnp.float32),
    mesh=_mesh,
    scratch_types=[
        pltpu.VMEM((DEG_BASE + 1, CHUNK), jnp.int32),
        pltpu.VMEM((CHUNK,), jnp.float32),
        pltpu.VMEM((ROWS_PER_TILE,), jnp.float32),
        pltpu.VMEM_SHARED((N_P,), jnp.float32),
    ],
    compiler_params=pltpu.CompilerParams(use_tc_tiling_on_sc=False),
)
def _sc_degree(ei_hbm, out_hbm, idx_v, ones_v, buf_v, acc_sh):
    c = lax.axis_index("c")
    s = lax.axis_index("s")
    wid = s * NC + c
    start = wid * DEG_BASE + jnp.minimum(wid, DEG_EXTRA)

    def fill16(i, _):
        ones_v[pl.ds(i * 16, 16)] = jnp.ones((16,), jnp.float32)
        return 0
    lax.fori_loop(0, CHUNK // 16, fill16, 0)

    def zero16(i, _):
        buf_v[pl.ds(i * 16, 16)] = jnp.zeros((16,), jnp.float32)
        return 0
    lax.fori_loop(0, ROWS_PER_TILE // 16, zero16, 0)

    pltpu.sync_copy(buf_v, acc_sh.at[pl.ds(s * ROWS_PER_TILE, ROWS_PER_TILE)])
    plsc.subcore_barrier()

    pltpu.sync_copy(ei_hbm.at[1, pl.ds(start, DEG_BASE)],
                    idx_v.at[pl.ds(0, DEG_BASE)])

    @pl.when(wid < DEG_EXTRA)
    def _():
        pltpu.sync_copy(ei_hbm.at[1, start + DEG_BASE], idx_v.at[DEG_BASE])

    def body(j, _):
        pltpu.sync_copy(ones_v, acc_sh.at[idx_v.at[j]], add=True)
        return 0
    lax.fori_loop(0, DEG_BASE, body, 0)

    @pl.when(wid < DEG_EXTRA)
    def _():
        pltpu.sync_copy(ones_v, acc_sh.at[idx_v.at[DEG_BASE]], add=True)

    plsc.subcore_barrier()
    rows = pl.ds(s * ROWS_PER_TILE, ROWS_PER_TILE)
    pltpu.sync_copy(acc_sh.at[rows], buf_v)
    pltpu.sync_copy(buf_v, out_hbm.at[c, rows])


def _fused_body(x_ref, w1_ref, b1_ref, wg_ref, d0_ref, d1_ref,
                glo_ref, ghi_ref, dinv_ref):
    h = jnp.maximum(_mm_nt(x_ref[...], w1_ref[...]) + b1_ref[...], 0.0)
    hw = _mm_nt(h, wg_ref[...])
    deg = d0_ref[...] + d1_ref[...] + 1.0
    dinv = lax.rsqrt(deg)
    dinv_ref[...] = dinv
    g = hw * dinv
    gb = g.astype(jnp.bfloat16)
    glo_ref[...] = gb[:, :DH]
    ghi_ref[...] = gb[:, DH:]


def _tc_fused(xp, W1, b1_2d, Wg, deg0, deg1):
    return pl.pallas_call(
        _fused_body,
        grid=(N_P // BLK,),
        in_specs=[
            pl.BlockSpec((BLK, D), lambda i: (i, 0)),
            pl.BlockSpec((D, D), lambda i: (0, 0)),
            pl.BlockSpec((1, D), lambda i: (0, 0)),
            pl.BlockSpec((D, D), lambda i: (0, 0)),
            pl.BlockSpec((BLK, 1), lambda i: (i, 0)),
            pl.BlockSpec((BLK, 1), lambda i: (i, 0)),
        ],
        out_specs=[
            pl.BlockSpec((BLK, DH), lambda i: (i, 0)),
            pl.BlockSpec((BLK, DH), lambda i: (i, 0)),
            pl.BlockSpec((BLK, 1), lambda i: (i, 0)),
        ],
        out_shape=[
            jax.ShapeDtypeStruct((N_P, DH), jnp.bfloat16),
            jax.ShapeDtypeStruct((N_P, DH), jnp.bfloat16),
            jax.ShapeDtypeStruct((N_P, 1), jnp.float32),
        ],
    )(xp, W1, b1_2d, Wg, deg0, deg1)


@functools.partial(
    pl.kernel,
    out_type=[
        jax.ShapeDtypeStruct((N_P, DH), jnp.bfloat16),
        jax.ShapeDtypeStruct((N_P, DH), jnp.bfloat16),
    ],
    mesh=_mesh,
    scratch_types=[
        pltpu.VMEM((SCAT_BASE + 1, CHUNK), jnp.int32),
        pltpu.VMEM((SCAT_BASE + 1, CHUNK), jnp.int32),
        pltpu.VMEM((NBUF, CHUNK, DH), jnp.bfloat16),
        pltpu.VMEM_SHARED((N_P, DH), jnp.bfloat16),
        [pltpu.SemaphoreType.DMA] * NBUF,
        [pltpu.SemaphoreType.DMA] * NBUF,
    ],
    compiler_params=pltpu.CompilerParams(use_tc_tiling_on_sc=False),
)
def _sc_scatter(ei_hbm, glo_hbm, ghi_hbm, outlo_hbm, outhi_hbm,
                src_v, dst_v, ring, acc_sh, gsems, ssems):
    c = lax.axis_index("c")
    s = lax.axis_index("s")
    start = s * SCAT_BASE + jnp.minimum(s, SCAT_EXTRA)
    cnt = SCAT_BASE + jnp.where(s < SCAT_EXTRA, 1, 0)

    def zero_row(i, _):
        ring[0, i // (DH // 32), pl.ds((i % (DH // 32)) * 32, 32)] = (
            jnp.zeros((32,), jnp.bfloat16))
        return 0
    lax.fori_loop(0, CHUNK * (DH // 32), zero_row, 0)

    def zero_acc(k, _):
        pltpu.sync_copy(ring.at[0],
                        acc_sh.at[pl.ds(s * ROWS_PER_TILE + k * CHUNK, CHUNK)])
        return 0
    lax.fori_loop(0, ROWS_PER_TILE // CHUNK, zero_acc, 0)
    plsc.subcore_barrier()

    pltpu.sync_copy(ei_hbm.at[0, pl.ds(start, SCAT_BASE)],
                    src_v.at[pl.ds(0, SCAT_BASE)])
    pltpu.sync_copy(ei_hbm.at[1, pl.ds(start, SCAT_BASE)],
                    dst_v.at[pl.ds(0, SCAT_BASE)])

    @pl.when(s < SCAT_EXTRA)
    def _():
        pltpu.sync_copy(ei_hbm.at[0, start + SCAT_BASE], src_v.at[SCAT_BASE])
        pltpu.sync_copy(ei_hbm.at[1, start + SCAT_BASE], dst_v.at[SCAT_BASE])

    def gather(g_hbm):
        for k in range(NBUF - 1):
            pltpu.async_copy(g_hbm.at[src_v.at[k]], ring.at[k], gsems[k])

        def body(jj, _):
            for k in range(NBUF):
                j = jj * NBUF + k
                m = (k + NBUF - 1) % NBUF
                pltpu.make_async_copy(g_hbm.at[src_v.at[j]], ring.at[k],
                                      gsems[k]).wait()

                @pl.when(jnp.logical_and(j >= 1, j + NBUF - 1 < cnt))
                def _():
                    pltpu.make_async_copy(ring.at[m],
                                          acc_sh.at[dst_v.at[j]],
                                          ssems[m]).wait()

                @pl.when(j + NBUF - 1 < cnt)
                def _():
                    pltpu.async_copy(g_hbm.at[src_v.at[j + NBUF - 1]],
                                     ring.at[m], gsems[m])

                pltpu.async_copy(ring.at[k], acc_sh.at[dst_v.at[j]],
                                 ssems[k], add=True)
            return 0
        lax.fori_loop(0, SCAT_BASE // NBUF, body, 0)

        @pl.when(cnt > SCAT_BASE)
        def _():
            pltpu.make_async_copy(g_hbm.at[src_v.at[SCAT_BASE]], ring.at[0],
                                  gsems[0]).wait()
            pltpu.async_copy(ring.at[0], acc_sh.at[dst_v.at[SCAT_BASE]],
                             ssems[0], add=True)

        for k in range(NBUF):
            pltpu.make_async_copy(ring.at[k], acc_sh.at[dst_v.at[0]],
                                  ssems[k]).wait()

    @pl.when(c == 0)
    def _():
        gather(glo_hbm)

    @pl.when(c == 1)
    def _():
        gather(ghi_hbm)

    plsc.subcore_barrier()

    def writeout(out_hbm):
        def wo(k, _):
            off = s * ROWS_PER_TILE + k * CHUNK
            pltpu.sync_copy(acc_sh.at[pl.ds(off, CHUNK)], ring.at[0])
            pltpu.sync_copy(ring.at[0], out_hbm.at[pl.ds(off, CHUNK)])
            return 0
        lax.fori_loop(0, ROWS_PER_TILE // CHUNK, wo, 0)

    @pl.when(c == 0)
    def _():
        writeout(outlo_hbm)

    @pl.when(c == 1)
    def _():
        writeout(outhi_hbm)


def _out_body(alo_ref, ahi_ref, glo_ref, ghi_ref, dinv_ref, bg_ref,
              w2_ref, b2_ref, o_ref):
    dinv = dinv_ref[...]
    alo = alo_ref[...].astype(jnp.float32)
    ahi = ahi_ref[...].astype(jnp.float32)
    glo = glo_ref[...].astype(jnp.float32)
    ghi = ghi_ref[...].astype(jnp.float32)
    conv = jnp.concatenate(
        [(alo + glo) * dinv,
         (ahi + ghi) * dinv], axis=1) + bg_ref[...]
    h2 = jnp.maximum(conv, 0.0)
    o_ref[...] = _mm_nt(h2, w2_ref[...]) + b2_ref[...]


OBLK = 2000


def _tc_out(alo, ahi, glo, ghi, dinv, bg_2d, W2, b2_2d):
    return pl.pallas_call(
        _out_body,
        grid=(N_REAL // OBLK,),
        in_specs=[
            pl.BlockSpec((OBLK, DH), lambda i: (i, 0)),
            pl.BlockSpec((OBLK, DH), lambda i: (i, 0)),
            pl.BlockSpec((OBLK, DH), lambda i: (i, 0)),
            pl.BlockSpec((OBLK, DH), lambda i: (i, 0)),
            pl.BlockSpec((OBLK, 1), lambda i: (i, 0)),
            pl.BlockSpec((1, D), lambda i: (0, 0)),
            pl.BlockSpec((D, D), lambda i: (0, 0)),
            pl.BlockSpec((1, D), lambda i: (0, 0)),
        ],
        out_specs=pl.BlockSpec((OBLK, D), lambda i: (i, 0)),
        out_shape=jax.ShapeDtypeStruct((N_REAL, D), jnp.float32),
    )(alo, ahi, glo, ghi, dinv, bg_2d, W2, b2_2d)


def kernel(x, edge_index, W1, b1, Wg, bg, W2, b2):
    ei3 = edge_index.astype(jnp.int32).reshape(2, N_CHUNKS, CHUNK)

    b1_2d = b1.reshape(1, D)
    bg_2d = bg.reshape(1, D)
    b2_2d = b2.reshape(1, D)

    deg_partials = _sc_degree(ei3)
    deg0 = deg_partials[0].reshape(N_P, 1)
    deg1 = deg_partials[1].reshape(N_P, 1)

    glo, ghi, dinv = _tc_fused(x, W1, b1_2d, Wg, deg0, deg1)

    alo, ahi = _sc_scatter(ei3, glo, ghi)

    return _tc_out(alo, ahi, glo, ghi, dinv, bg_2d, W2, b2_2d)


# speedup vs baseline: 1.1135x; 1.0858x over previous
import functools

import jax
import jax.numpy as jnp
from jax import lax
from jax.experimental import pallas as pl
from jax.experimental.pallas import tpu as pltpu
from jax.experimental.pallas import tpu_sc as plsc

N_REAL = 10000
D = 128
DH = D // 2
E_REAL = 320000

NC = 2
NS = 16
NW = NC * NS
CHUNK = 128
N_CHUNKS = E_REAL // CHUNK
DEG_BASE = N_CHUNKS // NW
DEG_EXTRA = N_CHUNKS - DEG_BASE * NW
SCAT_BASE = N_CHUNKS // NS
SCAT_EXTRA = N_CHUNKS - SCAT_BASE * NS
N_P = 10240
ROWS_PER_TILE = N_P // NS
BLK = 2048
NBUF = 6

_mesh = plsc.VectorSubcoreMesh(core_axis_name="c", subcore_axis_name="s")


def _mm_nt(a, b):
    return lax.dot_general(a, b, (((1,), (1,)), ((), ())),
                           preferred_element_type=jnp.float32)


@functools.partial(
    pl.kernel,
    out_type=jax.ShapeDtypeStruct((NC, N_P), jnp.float32),
    mesh=_mesh,
    scratch_types=[
        pltpu.VMEM((DEG_BASE + 1, CHUNK), jnp.int32),
        pltpu.VMEM((CHUNK,), jnp.float32),
        pltpu.VMEM((ROWS_PER_TILE,), jnp.float32),
        pltpu.VMEM_SHARED((N_P,), jnp.float32),
        pltpu.SemaphoreType.DMA,
    ],
    compiler_params=pltpu.CompilerParams(use_tc_tiling_on_sc=False),
)
def _sc_degree(ei_hbm, out_hbm, idx_v, ones_v, buf_v, acc_sh, dsem):
    c = lax.axis_index("c")
    s = lax.axis_index("s")
    wid = s * NC + c
    start = wid * DEG_BASE + jnp.minimum(wid, DEG_EXTRA)

    def fill16(i, _):
        ones_v[pl.ds(i * 16, 16)] = jnp.ones((16,), jnp.float32)
        return 0
    lax.fori_loop(0, CHUNK // 16, fill16, 0)

    def zero16(i, _):
        buf_v[pl.ds(i * 16, 16)] = jnp.zeros((16,), jnp.float32)
        return 0
    lax.fori_loop(0, ROWS_PER_TILE // 16, zero16, 0)

    pltpu.sync_copy(buf_v, acc_sh.at[pl.ds(s * ROWS_PER_TILE, ROWS_PER_TILE)])
    plsc.subcore_barrier()

    pltpu.sync_copy(ei_hbm.at[1, pl.ds(start, DEG_BASE)],
                    idx_v.at[pl.ds(0, DEG_BASE)])

    @pl.when(wid < DEG_EXTRA)
    def _():
        pltpu.sync_copy(ei_hbm.at[1, start + DEG_BASE], idx_v.at[DEG_BASE])

    def body(j, _):
        pltpu.async_copy(ones_v, acc_sh.at[idx_v.at[j]], dsem, add=True)
        return 0
    lax.fori_loop(0, DEG_BASE, body, 0)

    @pl.when(wid < DEG_EXTRA)
    def _():
        pltpu.async_copy(ones_v, acc_sh.at[idx_v.at[DEG_BASE]], dsem, add=True)

    def drain(j, _):
        pltpu.make_async_copy(ones_v, acc_sh.at[idx_v.at[0]], dsem).wait()
        return 0
    lax.fori_loop(0, DEG_BASE, drain, 0)

    @pl.when(wid < DEG_EXTRA)
    def _():
        pltpu.make_async_copy(ones_v, acc_sh.at[idx_v.at[0]], dsem).wait()

    plsc.subcore_barrier()
    rows = pl.ds(s * ROWS_PER_TILE, ROWS_PER_TILE)
    pltpu.sync_copy(acc_sh.at[rows], buf_v)
    pltpu.sync_copy(buf_v, out_hbm.at[c, rows])


def _fused_body(x_ref, w1_ref, b1_ref, wg_ref, dt_ref,
                glo_ref, ghi_ref, dinv_ref):
    h = jnp.maximum(_mm_nt(x_ref[...], w1_ref[...]) + b1_ref[...], 0.0)
    hw = _mm_nt(h, wg_ref[...])
    deg = dt_ref[:, 0:1] + dt_ref[:, 1:2] + 1.0
    dinv = lax.rsqrt(deg)
    dinv_ref[...] = dinv
    g = hw * dinv
    gb = g.astype(jnp.bfloat16)
    glo_ref[...] = gb[:, :DH]
    ghi_ref[...] = gb[:, DH:]


def _tc_fused(xp, W1, b1_2d, Wg, degt):
    return pl.pallas_call(
        _fused_body,
        grid=(N_P // BLK,),
        in_specs=[
            pl.BlockSpec((BLK, D), lambda i: (i, 0)),
            pl.BlockSpec((D, D), lambda i: (0, 0)),
            pl.BlockSpec((1, D), lambda i: (0, 0)),
            pl.BlockSpec((D, D), lambda i: (0, 0)),
            pl.BlockSpec((BLK, 2), lambda i: (i, 0)),
        ],
        out_specs=[
            pl.BlockSpec((BLK, DH), lambda i: (i, 0)),
            pl.BlockSpec((BLK, DH), lambda i: (i, 0)),
            pl.BlockSpec((BLK, 1), lambda i: (i, 0)),
        ],
        out_shape=[
            jax.ShapeDtypeStruct((N_P, DH), jnp.bfloat16),
            jax.ShapeDtypeStruct((N_P, DH), jnp.bfloat16),
            jax.ShapeDtypeStruct((N_P, 1), jnp.float32),
        ],
    )(xp, W1, b1_2d, Wg, degt)


@functools.partial(
    pl.kernel,
    out_type=[
        jax.ShapeDtypeStruct((N_P, DH), jnp.bfloat16),
        jax.ShapeDtypeStruct((N_P, DH), jnp.bfloat16),
    ],
    mesh=_mesh,
    scratch_types=[
        pltpu.VMEM((SCAT_BASE + 1, CHUNK), jnp.int32),
        pltpu.VMEM((SCAT_BASE + 1, CHUNK), jnp.int32),
        pltpu.VMEM((NBUF, CHUNK, DH), jnp.bfloat16),
        pltpu.VMEM_SHARED((N_P, DH), jnp.bfloat16),
        [pltpu.SemaphoreType.DMA] * NBUF,
        [pltpu.SemaphoreType.DMA] * NBUF,
    ],
    compiler_params=pltpu.CompilerParams(use_tc_tiling_on_sc=False),
)
def _sc_scatter(ei_hbm, glo_hbm, ghi_hbm, outlo_hbm, outhi_hbm,
                src_v, dst_v, ring, acc_sh, gsems, ssems):
    c = lax.axis_index("c")
    s = lax.axis_index("s")
    start = s * SCAT_BASE + jnp.minimum(s, SCAT_EXTRA)
    cnt = SCAT_BASE + jnp.where(s < SCAT_EXTRA, 1, 0)

    def zero_row(i, _):
        ring[0, i // (DH // 32), pl.ds((i % (DH // 32)) * 32, 32)] = (
            jnp.zeros((32,), jnp.bfloat16))
        return 0
    lax.fori_loop(0, CHUNK * (DH // 32), zero_row, 0)

    def zero_acc(k, _):
        pltpu.sync_copy(ring.at[0],
                        acc_sh.at[pl.ds(s * ROWS_PER_TILE + k * CHUNK, CHUNK)])
        return 0
    lax.fori_loop(0, ROWS_PER_TILE // CHUNK, zero_acc, 0)
    plsc.subcore_barrier()

    pltpu.sync_copy(ei_hbm.at[0, pl.ds(start, SCAT_BASE)],
                    src_v.at[pl.ds(0, SCAT_BASE)])
    pltpu.sync_copy(ei_hbm.at[1, pl.ds(start, SCAT_BASE)],
                    dst_v.at[pl.ds(0, SCAT_BASE)])

    @pl.when(s < SCAT_EXTRA)
    def _():
        pltpu.sync_copy(ei_hbm.at[0, start + SCAT_BASE], src_v.at[SCAT_BASE])
        pltpu.sync_copy(ei_hbm.at[1, start + SCAT_BASE], dst_v.at[SCAT_BASE])

    def gather(g_hbm):
        for k in range(NBUF - 1):
            pltpu.async_copy(g_hbm.at[src_v.at[k]], ring.at[k], gsems[k])

        def body(jj, _):
            for k in range(NBUF):
                j = jj * NBUF + k
                m = (k + NBUF - 1) % NBUF
                pltpu.make_async_copy(g_hbm.at[src_v.at[j]], ring.at[k],
                                      gsems[k]).wait()

                @pl.when(jnp.logical_and(j >= 1, j + NBUF - 1 < cnt))
                def _():
                    pltpu.make_async_copy(ring.at[m],
                                          acc_sh.at[dst_v.at[j]],
                                          ssems[m]).wait()

                @pl.when(j + NBUF - 1 < cnt)
                def _():
                    pltpu.async_copy(g_hbm.at[src_v.at[j + NBUF - 1]],
                                     ring.at[m], gsems[m])

                pltpu.async_copy(ring.at[k], acc_sh.at[dst_v.at[j]],
                                 ssems[k], add=True)
            return 0
        lax.fori_loop(0, SCAT_BASE // NBUF, body, 0)

        @pl.when(cnt > SCAT_BASE)
        def _():
            pltpu.make_async_copy(g_hbm.at[src_v.at[SCAT_BASE]], ring.at[0],
                                  gsems[0]).wait()
            pltpu.async_copy(ring.at[0], acc_sh.at[dst_v.at[SCAT_BASE]],
                             ssems[0], add=True)

        for k in range(NBUF):
            pltpu.make_async_copy(ring.at[k], acc_sh.at[dst_v.at[0]],
                                  ssems[k]).wait()

    @pl.when(c == 0)
    def _():
        gather(glo_hbm)

    @pl.when(c == 1)
    def _():
        gather(ghi_hbm)

    plsc.subcore_barrier()

    def writeout(out_hbm):
        def wo(k, _):
            off = s * ROWS_PER_TILE + k * CHUNK
            pltpu.sync_copy(acc_sh.at[pl.ds(off, CHUNK)], ring.at[0])
            pltpu.sync_copy(ring.at[0], out_hbm.at[pl.ds(off, CHUNK)])
            return 0
        lax.fori_loop(0, ROWS_PER_TILE // CHUNK, wo, 0)

    @pl.when(c == 0)
    def _():
        writeout(outlo_hbm)

    @pl.when(c == 1)
    def _():
        writeout(outhi_hbm)


def _out_body(alo_ref, ahi_ref, glo_ref, ghi_ref, dinv_ref, bg_ref,
              w2_ref, b2_ref, o_ref):
    dinv = dinv_ref[...]
    alo = alo_ref[...].astype(jnp.float32)
    ahi = ahi_ref[...].astype(jnp.float32)
    glo = glo_ref[...].astype(jnp.float32)
    ghi = ghi_ref[...].astype(jnp.float32)
    conv = jnp.concatenate(
        [(alo + glo) * dinv,
         (ahi + ghi) * dinv], axis=1) + bg_ref[...]
    h2 = jnp.maximum(conv, 0.0)
    o_ref[...] = _mm_nt(h2, w2_ref[...]) + b2_ref[...]


OBLK = 2000


def _tc_out(alo, ahi, glo, ghi, dinv, bg_2d, W2, b2_2d):
    return pl.pallas_call(
        _out_body,
        grid=(N_REAL // OBLK,),
        in_specs=[
            pl.BlockSpec((OBLK, DH), lambda i: (i, 0)),
            pl.BlockSpec((OBLK, DH), lambda i: (i, 0)),
            pl.BlockSpec((OBLK, DH), lambda i: (i, 0)),
            pl.BlockSpec((OBLK, DH), lambda i: (i, 0)),
            pl.BlockSpec((OBLK, 1), lambda i: (i, 0)),
            pl.BlockSpec((1, D), lambda i: (0, 0)),
            pl.BlockSpec((D, D), lambda i: (0, 0)),
            pl.BlockSpec((1, D), lambda i: (0, 0)),
        ],
        out_specs=pl.BlockSpec((OBLK, D), lambda i: (i, 0)),
        out_shape=jax.ShapeDtypeStruct((N_REAL, D), jnp.float32),
    )(alo, ahi, glo, ghi, dinv, bg_2d, W2, b2_2d)


def kernel(x, edge_index, W1, b1, Wg, bg, W2, b2):
    ei3 = edge_index.astype(jnp.int32).reshape(2, N_CHUNKS, CHUNK)

    b1_2d = b1.reshape(1, D)
    bg_2d = bg.reshape(1, D)
    b2_2d = b2.reshape(1, D)

    degt = _sc_degree(ei3).T

    glo, ghi, dinv = _tc_fused(x, W1, b1_2d, Wg, degt)

    alo, ahi = _sc_scatter(ei3, glo, ghi)

    return _tc_out(alo, ahi, glo, ghi, dinv, bg_2d, W2, b2_2d)
